# Initial kernel scaffold; baseline (speedup 1.0000x reference)
import jax
import jax.numpy as jnp
from jax.experimental import pallas as pl


def kernel(global_features, curr_positions, target_positions, local_features, params):
    raise NotImplementedError("write your pallas kernel here")


# speedup vs baseline: 40.9038x; 40.9038x over previous
import functools

import jax
import jax.numpy as jnp
from jax import lax
from jax.experimental import pallas as pl
from jax.experimental.pallas import tpu as pltpu
from jax.experimental.pallas import tpu_sc as plsc

B = 4
A = 2048
H = 32
W = 32
C = 32
FOV = 11
N = B * A
T = B * H * W
DILS = (1, 1, 1, 2, 4, 8)
PADB = FOV // 2

NC = 2
NS = 16
NW = NC * NS
APW = N // NW
CHUNK = 128


def _wmat(w):
    return jnp.transpose(w, (2, 3, 1, 0)).reshape(9 * C, C).astype(jnp.bfloat16)


def _conv_body(relu, so, d, xp_ref, wm_ref, bias_ref, out_ref, cols_ref):
    for ky in range(3):
        for kx in range(3):
            t = ky * 3 + kx
            tap = xp_ref[0, ky * d:ky * d + so, kx * d:kx * d + so, :]
            cols_ref[:, t * C:(t + 1) * C] = tap.reshape(so * so, C)
    y = jnp.dot(cols_ref[...], wm_ref[...], preferred_element_type=jnp.float32)
    y = y + bias_ref[...]
    if relu:
        y = jnp.maximum(y, 0.0)
    out_ref[0] = y


@functools.partial(jax.jit, static_argnames=("so", "d", "relu"))
def _conv_tc(xp_bf16, wm, bias, so, d, relu):
    sp = so + 2 * d
    body = functools.partial(_conv_body, relu, so, d)
    return pl.pallas_call(
        body,
        grid=(B,),
        in_specs=[
            pl.BlockSpec((1, sp, sp, C), lambda b: (b, 0, 0, 0)),
            pl.BlockSpec((9 * C, C), lambda b: (0, 0)),
            pl.BlockSpec((1, C), lambda b: (0, 0)),
        ],
        out_specs=pl.BlockSpec((1, so * so, C), lambda b: (b, 0, 0)),
        out_shape=jax.ShapeDtypeStruct((B, so * so, C), jnp.float32),
        scratch_shapes=[pltpu.VMEM((so * so, 9 * C), jnp.bfloat16)],
    )(xp_bf16, wm, bias)


_SC_MESH = plsc.VectorSubcoreMesh(core_axis_name="c", subcore_axis_name="s",
                                  num_cores=NC, num_subcores=NS)


@functools.partial(
    pl.kernel,
    out_type=jax.ShapeDtypeStruct((N, C), jnp.float32),
    mesh=_SC_MESH,
    compiler_params=pltpu.CompilerParams(use_tc_tiling_on_sc=False),
    scratch_types=[
        pltpu.VMEM((CHUNK,), jnp.int32),
        pltpu.VMEM((CHUNK, C), jnp.float32),
        pltpu.SemaphoreType.DMA,
    ],
)
def _sc_gather(ytab_hbm, fidx_hbm, out_hbm, idx_v, row_v, sem):
    cid = lax.axis_index("c")
    sid = lax.axis_index("s")
    wid = sid * NC + cid
    for j in range(APW // CHUNK):
        off = wid * APW + j * CHUNK
        pltpu.sync_copy(fidx_hbm.at[pl.ds(off, CHUNK)], idx_v)
        pltpu.async_copy(ytab_hbm.at[idx_v], row_v, sem).wait()
        pltpu.sync_copy(row_v, out_hbm.at[pl.ds(off, CHUNK)])


def _bn(x, g, b):
    scale = g / jnp.sqrt(jnp.asarray(1.0 + 1e-5, dtype=x.dtype))
    if x.ndim == 4:
        return x * scale[None, :, None, None] + b[None, :, None, None]
    return x * scale + b


def _ln(x, w, b, axes):
    m = jnp.mean(x, axis=axes, keepdims=True)
    v = jnp.var(x, axis=axes, keepdims=True)
    return (x - m) / jnp.sqrt(v + 1e-5) * w + b


@jax.jit
def _run(gf, curr, lf, p):
    g = _bn(gf, p["gp_bn_g"], p["gp_bn_b"])
    g = jax.lax.conv_general_dilated(
        g, p["gp_w"], (1, 1), ((0, 0), (0, 0)),
        dimension_numbers=("NCHW", "OIHW", "NCHW")) + p["gp_b"][None, :, None, None]
    l = _bn(lf, p["lp_bn_g"], p["lp_bn_b"])
    l = l @ p["lp_w"].T + p["lp_b"]
    idx0 = jnp.repeat(jnp.arange(B), A)
    x = g.at[idx0, :, curr[:, 0], curr[:, 1]].add(l)

    for i in range(6):
        d = DILS[i]
        xn = _ln(x, p["gb_ln_w"][i], p["gb_ln_b"][i], (1, 2, 3))
        xp = jnp.transpose(xn.astype(jnp.bfloat16), (0, 2, 3, 1))
        xp = jnp.pad(xp, ((0, 0), (d, d), (d, d), (0, 0)))
        y = _conv_tc(xp, _wmat(p["gb_conv_w"][i]),
                     p["gb_conv_b"][i][None, :], H, d, True)
        x = jnp.transpose(y.reshape(B, H, W, C), (0, 3, 1, 2))

    x = _ln(x, p["gb_lnf_w"], p["gb_lnf_b"], (-1,))
    y = jnp.transpose(x, (0, 2, 3, 1))
    y = jnp.pad(y, ((0, 0), (PADB, PADB), (PADB, PADB), (0, 0)),
                constant_values=-1.0)
    for j in range(5):
        s = H + 2 * PADB - 2 * j
        scale = (p["bb_bn_g"][j] / jnp.sqrt(jnp.asarray(1.0 + 1e-5, jnp.float32)))
        yb = (y * scale[None, None, None, :]
              + p["bb_bn_b"][j][None, None, None, :]).astype(jnp.bfloat16)
        z = _conv_tc(yb, _wmat(p["bb_conv_w"][j]),
                     p["bb_conv_b"][j][None, :], s - 2, 1, j < 4)
        y = z.reshape(B, s - 2, s - 2, C)

    ytab = y.reshape(T, C)
    fidx = (idx0 * (H * W) + curr[:, 0] * W + curr[:, 1]).astype(jnp.int32)
    return _sc_gather(ytab, fidx)


def kernel(global_features, curr_positions, target_positions, local_features,
           params):
    return _run(global_features, curr_positions, local_features, params)


# speedup vs baseline: 50.6057x; 1.2372x over previous
import functools

import jax
import jax.numpy as jnp
from jax import lax
from jax.experimental import pallas as pl
from jax.experimental.pallas import tpu as pltpu
from jax.experimental.pallas import tpu_sc as plsc

B = 4
A = 2048
H = 32
W = 32
C = 32
FOV = 11
N = B * A
T = B * H * W
DILS = (1, 1, 1, 2, 4, 8)
PADB = FOV // 2

NC = 2
NS = 16
NW = NC * NS
APW = N // NW
CHUNK = 128


def _wmat(w):
    return jnp.transpose(w, (2, 3, 1, 0)).reshape(9 * C, C).astype(jnp.bfloat16)


def _conv_body(relu, so, d, xp_ref, wm_ref, bias_ref, out_ref, cols_ref):
    for ky in range(3):
        for kx in range(3):
            t = ky * 3 + kx
            tap = xp_ref[0, ky * d:ky * d + so, kx * d:kx * d + so, :]
            cols_ref[:, t * C:(t + 1) * C] = tap.reshape(so * so, C)
    y = jnp.dot(cols_ref[...], wm_ref[...], preferred_element_type=jnp.float32)
    y = y + bias_ref[...]
    if relu:
        y = jnp.maximum(y, 0.0)
    out_ref[0] = y


@functools.partial(jax.jit, static_argnames=("so", "d", "relu"))
def _conv_tc(xp_bf16, wm, bias, so, d, relu):
    sp = so + 2 * d
    body = functools.partial(_conv_body, relu, so, d)
    return pl.pallas_call(
        body,
        grid=(B,),
        in_specs=[
            pl.BlockSpec((1, sp, sp, C), lambda b: (b, 0, 0, 0)),
            pl.BlockSpec((9 * C, C), lambda b: (0, 0)),
            pl.BlockSpec((1, C), lambda b: (0, 0)),
        ],
        out_specs=pl.BlockSpec((1, so * so, C), lambda b: (b, 0, 0)),
        out_shape=jax.ShapeDtypeStruct((B, so * so, C), jnp.float32),
        scratch_shapes=[pltpu.VMEM((so * so, 9 * C), jnp.bfloat16)],
    )(xp_bf16, wm, bias)


def _bb_body(ypad_ref, wm_ref, bias_ref, bns_ref, bnb_ref, out_ref,
             cols_ref, pb0, pb1, pb2, pb3):
    S0 = H + 2 * PADB
    pbufs = (pb0, pb1, pb2, pb3)
    src = None
    for j in range(5):
        s = S0 - 2 * j
        so = s - 2
        rows = so * so
        base = ypad_ref[0, 0:s, 0:s, :] if j == 0 else src[0:s, 0:s, :]
        srcv = base * bns_ref[j][None, None, :] + bnb_ref[j][None, None, :]
        for ky in range(3):
            for kx in range(3):
                t = ky * 3 + kx
                tap = srcv[ky:ky + so, kx:kx + so, :]
                cols_ref[0:rows, t * C:(t + 1) * C] = (
                    tap.reshape(rows, C).astype(jnp.bfloat16))
        y = jnp.dot(cols_ref[0:rows, :], wm_ref[j],
                    preferred_element_type=jnp.float32) + bias_ref[j][None, :]
        if j < 4:
            y = jnp.maximum(y, 0.0)
            pbufs[j][...] = y.reshape(so, so, C)
            src = pbufs[j]
    out_ref[0] = y


@jax.jit
def _bb_tc(ypad_bf, wm, bias, bns, bnb):
    S0 = H + 2 * PADB
    return pl.pallas_call(
        _bb_body,
        grid=(B,),
        in_specs=[
            pl.BlockSpec((1, S0, S0, C), lambda b: (b, 0, 0, 0)),
            pl.BlockSpec((5, 9 * C, C), lambda b: (0, 0, 0)),
            pl.BlockSpec((5, C), lambda b: (0, 0)),
            pl.BlockSpec((5, C), lambda b: (0, 0)),
            pl.BlockSpec((5, C), lambda b: (0, 0)),
        ],
        out_specs=pl.BlockSpec((1, H * W, C), lambda b: (b, 0, 0)),
        out_shape=jax.ShapeDtypeStruct((B, H * W, C), jnp.float32),
        scratch_shapes=[
            pltpu.VMEM(((S0 - 2) * (S0 - 2), 9 * C), jnp.bfloat16),
            pltpu.VMEM((40, 40, C), jnp.float32),
            pltpu.VMEM((38, 38, C), jnp.float32),
            pltpu.VMEM((36, 36, C), jnp.float32),
            pltpu.VMEM((34, 34, C), jnp.float32),
        ],
    )(ypad_bf, wm, bias, bns, bnb)


_SC_MESH = plsc.VectorSubcoreMesh(core_axis_name="c", subcore_axis_name="s",
                                  num_cores=NC, num_subcores=NS)


@functools.partial(
    pl.kernel,
    out_type=jax.ShapeDtypeStruct((N, C), jnp.float32),
    mesh=_SC_MESH,
    compiler_params=pltpu.CompilerParams(use_tc_tiling_on_sc=False),
    scratch_types=[
        pltpu.VMEM((CHUNK,), jnp.int32),
        pltpu.VMEM((CHUNK, C), jnp.float32),
        pltpu.SemaphoreType.DMA,
    ],
)
def _sc_gather(ytab_hbm, fidx_hbm, out_hbm, idx_v, row_v, sem):
    cid = lax.axis_index("c")
    sid = lax.axis_index("s")
    wid = sid * NC + cid
    for j in range(APW // CHUNK):
        off = wid * APW + j * CHUNK
        pltpu.sync_copy(fidx_hbm.at[pl.ds(off, CHUNK)], idx_v)
        pltpu.async_copy(ytab_hbm.at[idx_v], row_v, sem).wait()
        pltpu.sync_copy(row_v, out_hbm.at[pl.ds(off, CHUNK)])


def _bn(x, g, b):
    scale = g / jnp.sqrt(jnp.asarray(1.0 + 1e-5, dtype=x.dtype))
    if x.ndim == 4:
        return x * scale[None, :, None, None] + b[None, :, None, None]
    return x * scale + b


def _ln(x, w, b, axes):
    m = jnp.mean(x, axis=axes, keepdims=True)
    v = jnp.var(x, axis=axes, keepdims=True)
    return (x - m) / jnp.sqrt(v + 1e-5) * w + b


@jax.jit
def _run(gf, curr, lf, p):
    g = _bn(gf, p["gp_bn_g"], p["gp_bn_b"])
    g = jax.lax.conv_general_dilated(
        g, p["gp_w"], (1, 1), ((0, 0), (0, 0)),
        dimension_numbers=("NCHW", "OIHW", "NCHW")) + p["gp_b"][None, :, None, None]
    l = _bn(lf, p["lp_bn_g"], p["lp_bn_b"])
    l = l @ p["lp_w"].T + p["lp_b"]
    idx0 = jnp.repeat(jnp.arange(B), A)
    x = g.at[idx0, :, curr[:, 0], curr[:, 1]].add(l)

    for i in range(6):
        d = DILS[i]
        xn = _ln(x, p["gb_ln_w"][i], p["gb_ln_b"][i], (1, 2, 3))
        xp = jnp.transpose(xn.astype(jnp.bfloat16), (0, 2, 3, 1))
        xp = jnp.pad(xp, ((0, 0), (d, d), (d, d), (0, 0)))
        y = _conv_tc(xp, _wmat(p["gb_conv_w"][i]),
                     p["gb_conv_b"][i][None, :], H, d, True)
        x = jnp.transpose(y.reshape(B, H, W, C), (0, 3, 1, 2))

    x = _ln(x, p["gb_lnf_w"], p["gb_lnf_b"], (-1,))
    y = jnp.transpose(x, (0, 2, 3, 1))
    y = jnp.pad(y, ((0, 0), (PADB, PADB), (PADB, PADB), (0, 0)),
                constant_values=-1.0)
    rc = 1.0 / jnp.sqrt(jnp.asarray(1.0 + 1e-5, jnp.float32))
    bns = jnp.stack([g * rc for g in p["bb_bn_g"]])
    bnb = jnp.stack(p["bb_bn_b"])
    wm = jnp.stack([_wmat(w) for w in p["bb_conv_w"]])
    bias = jnp.stack(p["bb_conv_b"])
    y = _bb_tc(y, wm, bias, bns, bnb)

    ytab = y.reshape(T, C)
    fidx = (idx0 * (H * W) + curr[:, 0] * W + curr[:, 1]).astype(jnp.int32)
    return _sc_gather(ytab, fidx)


def kernel(global_features, curr_positions, target_positions, local_features,
           params):
    return _run(global_features, curr_positions, local_features, params)
